# baseline (speedup 1.0000x reference)
import functools

import jax
import jax.numpy as jnp
from jax.experimental import pallas as pl
from jax.experimental.pallas import tpu as pltpu

_B = 16384
_D = 64
_NCH = 8
_CH = _B // _NCH
_NBUF = 4
_NCP = 2


def _body(gu_ref, gi_ref, xui_ref, gamu_ref, gami_ref,
          u_buf, i_buf, xacc, sin_u, sin_i, scp_u, scp_i, sx):
    half = _B // _NCP

    def cp_hbm(src, dst, sem):
        return [pltpu.make_async_copy(src.at[pl.ds(j * half, half)],
                                      dst.at[pl.ds(j * half, half)],
                                      sem.at[j])
                for j in range(_NCP)]

    def cp_in(c, b):
        sl = pl.ds(c * _CH, _CH)
        return (pltpu.make_async_copy(gu_ref.at[sl], u_buf.at[b], sin_u.at[b]),
                pltpu.make_async_copy(gi_ref.at[sl], i_buf.at[b], sin_i.at[b]))

    for cp in cp_hbm(gu_ref, gamu_ref, scp_u) + cp_hbm(gi_ref, gami_ref, scp_i):
        cp.start()

    for c in range(_NBUF):
        for cp in cp_in(c, c % _NBUF):
            cp.start()

    for c in range(_NCH):
        b = c % _NBUF
        for cp in cp_in(c, b):
            cp.wait()
        xacc[pl.ds(c * _CH, _CH)] = jnp.sum(u_buf[b] * i_buf[b], axis=1)
        nxt = c + _NBUF
        if nxt < _NCH:
            for cp in cp_in(nxt, b):
                cp.start()

    xcp = pltpu.make_async_copy(xacc, xui_ref, sx)
    xcp.start()
    xcp.wait()

    for cp in cp_hbm(gu_ref, gamu_ref, scp_u) + cp_hbm(gi_ref, gami_ref, scp_i):
        cp.wait()


@jax.jit
def _uuii_tc(gu, gi):
    return pl.pallas_call(
        _body,
        in_specs=[
            pl.BlockSpec(memory_space=pl.MemorySpace.ANY),
            pl.BlockSpec(memory_space=pl.MemorySpace.ANY),
        ],
        out_specs=[
            pl.BlockSpec(memory_space=pl.MemorySpace.ANY),
            pl.BlockSpec(memory_space=pl.MemorySpace.ANY),
            pl.BlockSpec(memory_space=pl.MemorySpace.ANY),
        ],
        out_shape=[
            jax.ShapeDtypeStruct((_B,), jnp.float32),
            jax.ShapeDtypeStruct((_B, _D), jnp.float32),
            jax.ShapeDtypeStruct((_B, _D), jnp.float32),
        ],
        scratch_shapes=[
            pltpu.VMEM((_NBUF, _CH, _D), jnp.float32),
            pltpu.VMEM((_NBUF, _CH, _D), jnp.float32),
            pltpu.VMEM((_B,), jnp.float32),
            pltpu.SemaphoreType.DMA((_NBUF,)),
            pltpu.SemaphoreType.DMA((_NBUF,)),
            pltpu.SemaphoreType.DMA((_NCP,)),
            pltpu.SemaphoreType.DMA((_NCP,)),
            pltpu.SemaphoreType.DMA,
        ],
    )(gu, gi)


def kernel(gu, gi):
    xui, gamma_u, gamma_i = _uuii_tc(gu, gi)
    return (xui, gamma_u, gamma_i)


# speedup vs baseline: 13.9461x; 13.9461x over previous
import functools

import jax
import jax.numpy as jnp
from jax.experimental import pallas as pl
from jax.experimental.pallas import tpu as pltpu

_B = 16384
_D = 64
_NCH = 16
_CH = _B // _NCH
_NBUF = 8


def _body(gu_ref, gi_ref, xui_ref, gamu_ref, gami_ref,
          u_buf, i_buf, xacc, sin_u, sin_i, sout_u, sout_i, sx):

    def cp_in(c, b):
        sl = pl.ds(c * _CH, _CH)
        return (pltpu.make_async_copy(gu_ref.at[sl], u_buf.at[b], sin_u.at[b]),
                pltpu.make_async_copy(gi_ref.at[sl], i_buf.at[b], sin_i.at[b]))

    def cp_out(c, b):
        sl = pl.ds(c * _CH, _CH)
        return (pltpu.make_async_copy(u_buf.at[b], gamu_ref.at[sl], sout_u.at[b]),
                pltpu.make_async_copy(i_buf.at[b], gami_ref.at[sl], sout_i.at[b]))

    for c in range(_NBUF - 1):
        for cp in cp_in(c, c % _NBUF):
            cp.start()

    for c in range(_NCH):
        b = c % _NBUF
        for cp in cp_in(c, b):
            cp.wait()
        xacc[pl.ds(c * _CH, _CH)] = jnp.sum(u_buf[b] * i_buf[b], axis=1)
        for cp in cp_out(c, b):
            cp.start()
        nxt = c + (_NBUF - 1)
        if nxt < _NCH:
            nb = nxt % _NBUF
            prev = nxt - _NBUF
            if prev >= 0:
                for cp in cp_out(prev, nb):
                    cp.wait()
            for cp in cp_in(nxt, nb):
                cp.start()

    for c in range(_NCH - _NBUF, _NCH):
        for cp in cp_out(c, c % _NBUF):
            cp.wait()

    xcp = pltpu.make_async_copy(xacc, xui_ref, sx)
    xcp.start()
    xcp.wait()


@jax.jit
def _uuii_tc(gu, gi):
    return pl.pallas_call(
        _body,
        in_specs=[
            pl.BlockSpec(memory_space=pl.MemorySpace.ANY),
            pl.BlockSpec(memory_space=pl.MemorySpace.ANY),
        ],
        out_specs=[
            pl.BlockSpec(memory_space=pl.MemorySpace.ANY),
            pl.BlockSpec(memory_space=pl.MemorySpace.ANY),
            pl.BlockSpec(memory_space=pl.MemorySpace.ANY),
        ],
        out_shape=[
            jax.ShapeDtypeStruct((_B,), jnp.float32),
            jax.ShapeDtypeStruct((_B, _D), jnp.float32),
            jax.ShapeDtypeStruct((_B, _D), jnp.float32),
        ],
        scratch_shapes=[
            pltpu.VMEM((_NBUF, _CH, _D), jnp.float32),
            pltpu.VMEM((_NBUF, _CH, _D), jnp.float32),
            pltpu.VMEM((_B,), jnp.float32),
            pltpu.SemaphoreType.DMA((_NBUF,)),
            pltpu.SemaphoreType.DMA((_NBUF,)),
            pltpu.SemaphoreType.DMA((_NBUF,)),
            pltpu.SemaphoreType.DMA((_NBUF,)),
            pltpu.SemaphoreType.DMA,
        ],
    )(gu, gi)


def kernel(gu, gi):
    xui, gamma_u, gamma_i = _uuii_tc(gu, gi)
    return (xui, gamma_u, gamma_i)


# speedup vs baseline: 14.0138x; 1.0049x over previous
import jax
import jax.numpy as jnp
from jax.experimental import pallas as pl
from jax.experimental.pallas import tpu as pltpu

_B = 16384
_D = 64
_NCH = 16
_CH = _B // _NCH
_NBUF = 8


def _body(gu_ref, gi_ref, xui_ref, gamu_ref, gami_ref,
          u_buf, i_buf, xacc, sin_u, sin_i, sout_u, sout_i, sx):

    def cp_in(c, b):
        sl = pl.ds(c * _CH, _CH)
        return (pltpu.make_async_copy(gu_ref.at[sl], u_buf.at[b], sin_u.at[b]),
                pltpu.make_async_copy(gi_ref.at[sl], i_buf.at[b], sin_i.at[b]))

    def cp_out(c, b):
        sl = pl.ds(c * _CH, _CH)
        return (pltpu.make_async_copy(u_buf.at[b], gamu_ref.at[sl], sout_u.at[b]),
                pltpu.make_async_copy(i_buf.at[b], gami_ref.at[sl], sout_i.at[b]))

    for c in range(_NBUF - 1):
        for cp in cp_in(c, c % _NBUF):
            cp.start()

    for c in range(_NCH):
        b = c % _NBUF
        for cp in cp_in(c, b):
            cp.wait()
        xacc[pl.ds(c * _CH, _CH)] = jnp.sum(u_buf[b] * i_buf[b], axis=1)
        for cp in cp_out(c, b):
            cp.start()
        nxt = c + (_NBUF - 1)
        if nxt < _NCH:
            nb = nxt % _NBUF
            prev = nxt - _NBUF
            if prev >= 0:
                for cp in cp_out(prev, nb):
                    cp.wait()
            for cp in cp_in(nxt, nb):
                cp.start()

    for c in range(_NCH - _NBUF, _NCH):
        for cp in cp_out(c, c % _NBUF):
            cp.wait()

    xcp = pltpu.make_async_copy(xacc, xui_ref, sx)
    xcp.start()
    xcp.wait()


@jax.jit
def _uuii_tc(gu, gi):
    return pl.pallas_call(
        _body,
        in_specs=[
            pl.BlockSpec(memory_space=pl.MemorySpace.ANY),
            pl.BlockSpec(memory_space=pl.MemorySpace.ANY),
        ],
        out_specs=[
            pl.BlockSpec(memory_space=pl.MemorySpace.ANY),
            pl.BlockSpec(memory_space=pl.MemorySpace.ANY),
            pl.BlockSpec(memory_space=pl.MemorySpace.ANY),
        ],
        out_shape=[
            jax.ShapeDtypeStruct((_B,), jnp.float32),
            jax.ShapeDtypeStruct((_B, _D), jnp.float32),
            jax.ShapeDtypeStruct((_B, _D), jnp.float32),
        ],
        scratch_shapes=[
            pltpu.VMEM((_NBUF, _CH, _D), jnp.float32),
            pltpu.VMEM((_NBUF, _CH, _D), jnp.float32),
            pltpu.VMEM((_B,), jnp.float32),
            pltpu.SemaphoreType.DMA((_NBUF,)),
            pltpu.SemaphoreType.DMA((_NBUF,)),
            pltpu.SemaphoreType.DMA((_NBUF,)),
            pltpu.SemaphoreType.DMA((_NBUF,)),
            pltpu.SemaphoreType.DMA,
        ],
    )(gu, gi)


def kernel(gu, gi):
    xui, gamma_u, gamma_i = _uuii_tc(gu, gi)
    return (xui, gamma_u, gamma_i)
